# baseline (speedup 1.0000x reference)
import jax
import jax.numpy as jnp
from jax import lax
from jax.experimental import pallas as pl
from jax.experimental.pallas import tpu as pltpu
from jax.experimental.pallas import tpu_sc as plsc

B = 16384
K = 50
D = 32
TOK = 1000000
NC, NS, L = 2, 16, 16
NW = NC * NS

TBLK = 512
NBLK = 61
TPT = TBLK * NBLK
HALF = TOK // NC
REM = HALF - NS * TPT

SPW = B // NW
CS = 4
IDXC = CS * K
NCH = SPW // CS

_mesh = plsc.VectorSubcoreMesh(core_axis_name="c", subcore_axis_name="s")


def _tr_body(wflat, wrm_hbm, in0, in1, out0, out1, rot, semi0, semi1):
    cid = lax.axis_index("c")
    sid = lax.axis_index("s")
    base = cid * HALF + sid * TPT
    iota = lax.iota(jnp.int32, L)
    for k in range(L):
        rot[k, :] = (iota + k) & (L - 1)

    def start_in(tok0, ibuf, sem):
        for d in range(D):
            pltpu.async_copy(wflat.at[pl.ds(d * TOK + tok0, TBLK)],
                             ibuf.at[d], sem)

    def wait_in(ibuf, sem):
        for d in range(D):
            pltpu.make_async_copy(wflat.at[pl.ds(0, TBLK)], ibuf.at[d],
                                  sem).wait()

    def transpose(ibuf, obuf, ngrp):
        def kbody(k, carry):
            rk = rot[k, :]
            for h in range(D // L):
                dvec = rk + (h * L)
                for g in range(ngrp):
                    tvec = iota + (g * L)
                    v = plsc.load_gather(ibuf, [dvec, tvec])
                    plsc.store_scatter(obuf, [tvec, dvec], v)
            return carry

        lax.fori_loop(0, L, kbody, 0)

    def write_out(tok0, obuf):
        pltpu.sync_copy(obuf, wrm_hbm.at[pl.ds(tok0, TBLK)])

    start_in(base, in0, semi0)

    def pair(i, carry):
        b0 = i * 2
        start_in(base + (b0 + 1) * TBLK, in1, semi1)
        wait_in(in0, semi0)
        transpose(in0, out0, TBLK // L)
        write_out(base + b0 * TBLK, out0)
        start_in(base + (b0 + 2) * TBLK, in0, semi0)
        wait_in(in1, semi1)
        transpose(in1, out1, TBLK // L)
        write_out(base + (b0 + 1) * TBLK, out1)
        return carry

    lax.fori_loop(0, NBLK // 2, pair, 0)
    wait_in(in0, semi0)
    transpose(in0, out0, TBLK // L)
    write_out(base + (NBLK - 1) * TBLK, out0)

    @pl.when(sid == NS - 1)
    def _rem():
        rbase = cid * HALF + NS * TPT
        for d in range(D):
            pltpu.async_copy(wflat.at[pl.ds(d * TOK + rbase, REM)],
                             in1.at[d, pl.ds(0, REM)], semi1)
        for d in range(D):
            pltpu.make_async_copy(wflat.at[pl.ds(0, REM)],
                                  in1.at[d, pl.ds(0, REM)], semi1).wait()
        transpose(in1, out1, REM // L)
        pltpu.sync_copy(out1.at[pl.ds(0, REM)], wrm_hbm.at[pl.ds(rbase, REM)])


_transpose = pl.kernel(
    _tr_body,
    out_type=jax.ShapeDtypeStruct((TOK, D), jnp.float32),
    mesh=_mesh,
    scratch_types=[
        pltpu.VMEM((D, TBLK), jnp.float32),
        pltpu.VMEM((D, TBLK), jnp.float32),
        pltpu.VMEM((TBLK, D), jnp.float32),
        pltpu.VMEM((TBLK, D), jnp.float32),
        pltpu.VMEM((L, L), jnp.int32),
        pltpu.SemaphoreType.DMA,
        pltpu.SemaphoreType.DMA,
    ],
    compiler_params=pltpu.CompilerParams(use_tc_tiling_on_sc=False, needs_layout_passes=False),
)


def _gt_body(x_hbm, w2, out_hbm, idx_all, rows0, rows1, out_buf, sem0, sem1):
    wid = lax.axis_index("s") * NC + lax.axis_index("c")
    base = wid * (SPW * K)

    pltpu.sync_copy(x_hbm.at[pl.ds(base, SPW * K)], idx_all)

    def start(c, buf, sem):
        off = pl.multiple_of(c * IDXC, 8)
        pltpu.async_copy(w2.at[idx_all.at[pl.ds(off, IDXC)]], buf, sem)

    def wait(buf, sem):
        pltpu.make_async_copy(w2.at[idx_all.at[pl.ds(0, IDXC)]], buf, sem).wait()

    def reduce_chunk(buf, c):
        for s in range(CS):
            a0 = buf[s * K, 0:L]
            a1 = buf[s * K, L:D]
            for j in range(1, K):
                a0 = a0 + buf[s * K + j, 0:L]
                a1 = a1 + buf[s * K + j, L:D]
            row = c * CS + s
            out_buf[row, 0:L] = a0
            out_buf[row, L:D] = a1

    start(0, rows0, sem0)

    def pair(i, carry):
        c0 = i * 2
        start(c0 + 1, rows1, sem1)
        wait(rows0, sem0)
        reduce_chunk(rows0, c0)
        start(c0 + 2, rows0, sem0)
        wait(rows1, sem1)
        reduce_chunk(rows1, c0 + 1)
        return carry

    lax.fori_loop(0, NCH // 2 - 1, pair, 0)
    start(NCH - 1, rows1, sem1)
    wait(rows0, sem0)
    reduce_chunk(rows0, NCH - 2)
    wait(rows1, sem1)
    reduce_chunk(rows1, NCH - 1)

    pltpu.sync_copy(out_buf, out_hbm.at[pl.ds(wid * SPW, SPW)])


_emb_sum = pl.kernel(
    _gt_body,
    out_type=jax.ShapeDtypeStruct((B, D), jnp.float32),
    mesh=_mesh,
    scratch_types=[
        pltpu.VMEM((SPW * K,), jnp.int32),
        pltpu.VMEM((IDXC, D), jnp.float32),
        pltpu.VMEM((IDXC, D), jnp.float32),
        pltpu.VMEM((SPW, D), jnp.float32),
        pltpu.SemaphoreType.DMA,
        pltpu.SemaphoreType.DMA,
    ],
    compiler_params=pltpu.CompilerParams(use_tc_tiling_on_sc=False, needs_layout_passes=False),
)


def kernel(x, weight):
    wflat = lax.reshape(weight, (D * TOK,), dimensions=(1, 0))
    wrm = _transpose(wflat)
    return _emb_sum(x.reshape(-1), wrm)


# speedup vs baseline: 4.8864x; 4.8864x over previous
import jax
import jax.numpy as jnp
from jax import lax
from jax.experimental import pallas as pl
from jax.experimental.pallas import tpu as pltpu
from jax.experimental.pallas import tpu_sc as plsc

B = 16384
K = 50
D = 32
NC, NS, L = 2, 16, 16
NW = NC * NS
SPW = B // NW
CS = 4
IDXC = CS * K
NCH = SPW // CS

_mesh = plsc.VectorSubcoreMesh(core_axis_name="c", subcore_axis_name="s")


def _body(x_hbm, w_hbm, out_hbm, idx_all, rows0, rows1, out_buf, sem0, sem1):
    wid = lax.axis_index("s") * NC + lax.axis_index("c")
    base = wid * (SPW * K)

    pltpu.sync_copy(x_hbm.at[pl.ds(base, SPW * K)], idx_all)

    def start(c, buf, sem):
        off = pl.multiple_of(c * IDXC, 8)
        pltpu.async_copy(w_hbm.at[idx_all.at[pl.ds(off, IDXC)]], buf, sem)

    def wait(buf, sem):
        pltpu.make_async_copy(w_hbm.at[idx_all.at[pl.ds(0, IDXC)]], buf, sem).wait()

    def reduce_chunk(buf, c):
        for s in range(CS):
            a0 = buf[s * K, 0:L]
            a1 = buf[s * K, L:D]
            for j in range(1, K):
                a0 = a0 + buf[s * K + j, 0:L]
                a1 = a1 + buf[s * K + j, L:D]
            row = c * CS + s
            out_buf[row, 0:L] = a0
            out_buf[row, L:D] = a1

    start(0, rows0, sem0)

    def pair(i, carry):
        c0 = i * 2
        start(c0 + 1, rows1, sem1)
        wait(rows0, sem0)
        reduce_chunk(rows0, c0)
        start(c0 + 2, rows0, sem0)
        wait(rows1, sem1)
        reduce_chunk(rows1, c0 + 1)
        return carry

    lax.fori_loop(0, NCH // 2 - 1, pair, 0)
    start(NCH - 1, rows1, sem1)
    wait(rows0, sem0)
    reduce_chunk(rows0, NCH - 2)
    wait(rows1, sem1)
    reduce_chunk(rows1, NCH - 1)

    pltpu.sync_copy(out_buf, out_hbm.at[pl.ds(wid * SPW, SPW)])


_emb_sum = pl.kernel(
    _body,
    out_type=jax.ShapeDtypeStruct((B, D), jnp.float32),
    mesh=_mesh,
    scratch_types=[
        pltpu.VMEM((SPW * K,), jnp.int32),
        pltpu.VMEM((IDXC, D), jnp.float32),
        pltpu.VMEM((IDXC, D), jnp.float32),
        pltpu.VMEM((SPW, D), jnp.float32),
        pltpu.SemaphoreType.DMA,
        pltpu.SemaphoreType.DMA,
    ],
    compiler_params=pltpu.CompilerParams(use_tc_tiling_on_sc=False),
)


def kernel(x, weight):
    return _emb_sum(x.reshape(-1), weight)
